# baseline (speedup 1.0000x reference)
import jax
import jax.numpy as jnp
from jax import lax
from jax.experimental import pallas as pl
from jax.experimental.pallas import tpu as pltpu
from jax.experimental.pallas import tpu_sc as plsc

HIDDEN = 512
NCTX = 64
BATCH = 1024
TIME = 50

NC = 2
NS = 16
NW = NC * NS
BPW = BATCH // NW


def _sc_gather_body(table_hbm, idx_hbm, out_hbm, idx_v, rows_v, sem):
    wid = lax.axis_index("s") * NC + lax.axis_index("c")
    base = wid * BPW
    pltpu.sync_copy(idx_hbm.at[pl.ds(base, BPW)], idx_v)
    pltpu.async_copy(table_hbm.at[idx_v], rows_v, sem).wait()
    pltpu.sync_copy(rows_v, out_hbm.at[pl.ds(base, BPW)])


_SC_GATHER_CACHE = []


def _sc_gather(gain, idx):
    if not _SC_GATHER_CACHE:
        _SC_GATHER_CACHE.append(pl.kernel(
            _sc_gather_body,
            out_type=jax.ShapeDtypeStruct((BATCH, HIDDEN), jnp.float32),
            mesh=plsc.VectorSubcoreMesh(core_axis_name="c", subcore_axis_name="s"),
            scratch_types=[
                pltpu.VMEM((BPW,), jnp.int32),
                pltpu.VMEM((BPW, HIDDEN), jnp.float32),
                pltpu.SemaphoreType.DMA,
            ],
        ))
    return _SC_GATHER_CACHE[0](gain, idx)

TB = 2


def _scale_body(bg_ref, x_ref, o_ref, sp_ref):
    @pl.when(pl.program_id(0) == 0)
    def _():
        sp_ref[...] = jax.nn.softplus(bg_ref[...])

    o_ref[...] = x_ref[...] * sp_ref[...][None, :, :]


def _scale(xt, bg):
    return pl.pallas_call(
        _scale_body,
        grid=(TIME // TB,),
        in_specs=[
            pl.BlockSpec((BATCH, HIDDEN), lambda i: (0, 0)),
            pl.BlockSpec((TB, BATCH, HIDDEN), lambda i: (i, 0, 0)),
        ],
        out_specs=pl.BlockSpec((TB, BATCH, HIDDEN), lambda i: (i, 0, 0)),
        out_shape=jax.ShapeDtypeStruct((TIME, BATCH, HIDDEN), jnp.float32),
        scratch_shapes=[pltpu.VMEM((BATCH, HIDDEN), jnp.float32)],
    )(bg, xt)


@jax.jit
def kernel(x, context_idx, gain):
    bg = _sc_gather(gain, context_idx.astype(jnp.int32))
    xt = lax.transpose(x, (1, 0, 2))
    out_t = _scale(xt, bg)
    return lax.transpose(out_t, (1, 0, 2))


# speedup vs baseline: 1.2765x; 1.2765x over previous
import jax
import jax.numpy as jnp
from jax import lax
from jax.experimental import pallas as pl
from jax.experimental.pallas import tpu as pltpu
from jax.experimental.pallas import tpu_sc as plsc

HIDDEN = 512
NCTX = 64
BATCH = 1024
TIME = 50

NC = 2
NS = 16
NW = NC * NS
BPW = BATCH // NW


def _sc_gather_body(table_hbm, idx_hbm, out_hbm, idx_v, rows_v, sem):
    wid = lax.axis_index("s") * NC + lax.axis_index("c")
    base = wid * BPW
    pltpu.sync_copy(idx_hbm.at[pl.ds(base, BPW)], idx_v)
    pltpu.async_copy(table_hbm.at[idx_v], rows_v, sem).wait()
    pltpu.sync_copy(rows_v, out_hbm.at[pl.ds(base, BPW)])


_SC_GATHER_CACHE = []


def _sc_gather(gain, idx):
    if not _SC_GATHER_CACHE:
        _SC_GATHER_CACHE.append(pl.kernel(
            _sc_gather_body,
            out_type=jax.ShapeDtypeStruct((BATCH, HIDDEN), jnp.float32),
            mesh=plsc.VectorSubcoreMesh(core_axis_name="c", subcore_axis_name="s"),
            scratch_types=[
                pltpu.VMEM((BPW,), jnp.int32),
                pltpu.VMEM((BPW, HIDDEN), jnp.float32),
                pltpu.SemaphoreType.DMA,
            ],
        ))
    return _SC_GATHER_CACHE[0](gain, idx)

TB = 5


def _scale_body(bg_ref, x_ref, o_ref, sp_ref):
    @pl.when(pl.program_id(0) == 0)
    def _():
        sp_ref[...] = jax.nn.softplus(bg_ref[...])

    o_ref[...] = x_ref[...] * sp_ref[...][None, :, :]


def _scale(xt, bg):
    return pl.pallas_call(
        _scale_body,
        grid=(TIME // TB,),
        in_specs=[
            pl.BlockSpec((BATCH, HIDDEN), lambda i: (0, 0)),
            pl.BlockSpec((TB, BATCH, HIDDEN), lambda i: (i, 0, 0)),
        ],
        out_specs=pl.BlockSpec((TB, BATCH, HIDDEN), lambda i: (i, 0, 0)),
        out_shape=jax.ShapeDtypeStruct((TIME, BATCH, HIDDEN), jnp.float32),
        scratch_shapes=[pltpu.VMEM((BATCH, HIDDEN), jnp.float32)],
    )(bg, xt)


@jax.jit
def kernel(x, context_idx, gain):
    bg = jnp.take(gain, context_idx, axis=0)
    xt = lax.transpose(x, (1, 0, 2))
    out_t = _scale(xt, bg)
    return lax.transpose(out_t, (1, 0, 2))
